# baseline (speedup 1.0000x reference)
import functools

import jax
import jax.numpy as jnp
from jax import lax
from jax.experimental import pallas as pl
from jax.experimental.pallas import tpu as pltpu
from jax.experimental.pallas import tpu_sc as plsc

_NUM_TYPES = 128
_N = 100000
_L = 16
_G = _N // _L
_NC = 2
_NS = 16
_NW = _NC * _NS
_GPW = _G // _NW
_EXTRA = _G - _GPW * _NW
_CG = 16
_CH_ROWS = _CG * _L
_CH_ELEMS = _CH_ROWS * _NUM_TYPES
_NCHUNK = -(-(_GPW + 1) // _CG)
_TYPES_BUF = _NCHUNK * _CH_ROWS
_MAX_G0 = (_NW - 1) * _GPW + _EXTRA
_TYPES_PAD = ((_MAX_G0 * _L + _TYPES_BUF + 15) // 16) * 16


@functools.partial(
    pl.kernel,
    out_type=jax.ShapeDtypeStruct((_N * _NUM_TYPES,), jnp.float32),
    mesh=plsc.VectorSubcoreMesh(core_axis_name="c", subcore_axis_name="s"),
    scratch_types=[
        pltpu.VMEM((_TYPES_BUF,), jnp.int32),
        pltpu.VMEM((_CH_ELEMS,), jnp.float32),
        pltpu.VMEM((_CH_ELEMS,), jnp.float32),
        pltpu.SemaphoreType.DMA,
        pltpu.SemaphoreType.DMA,
    ],
    compiler_params=pltpu.CompilerParams(needs_layout_passes=False),
)
def _onehot_sc(types_hbm, out_hbm, t_v, buf_a, buf_b, sem_a, sem_b):
    cid = lax.axis_index("c")
    sid = lax.axis_index("s")
    wid = (sid * _NC + cid).astype(jnp.int32)
    g0 = wid * _GPW + jnp.minimum(wid, _EXTRA)
    gc = _GPW + (wid < _EXTRA).astype(jnp.int32)
    g_last = g0 + gc - _CG

    pltpu.sync_copy(types_hbm.at[pl.ds(g0 * _L, _TYPES_BUF)], t_v)

    zvec = jnp.zeros((_L,), jnp.float32)
    ones = jnp.ones((_L,), jnp.float32)
    lane_off = lax.iota(jnp.int32, _L) * _NUM_TYPES

    def zero_rows(i, _):
        buf_a[pl.ds(i * _L, _L)] = zvec
        buf_b[pl.ds(i * _L, _L)] = zvec
        return 0

    lax.fori_loop(0, _CH_ELEMS // _L, zero_rows, 0)

    def chunk_start(c):
        return jnp.minimum(g0 + c * _CG, g_last)

    def scatter_chunk(buf, cs, val):
        off = (cs - g0) * _L
        for g in range(_CG):
            tv = t_v[pl.ds(off + g * _L, _L)]
            plsc.store_scatter(buf, [lane_off + (g * _L * _NUM_TYPES) + tv],
                               val)

    bufs = (buf_a, buf_b)
    sems = (sem_a, sem_b)
    copies = [None, None]
    for c in range(_NCHUNK):
        b = c % 2
        if c >= 2:
            copies[b].wait()
            scatter_chunk(bufs[b], chunk_start(c - 2), zvec)
        cs = chunk_start(c)
        scatter_chunk(bufs[b], cs, ones)
        copies[b] = pltpu.async_copy(
            bufs[b], out_hbm.at[pl.ds(cs * _L * _NUM_TYPES, _CH_ELEMS)],
            sems[b])
    copies[(_NCHUNK - 1) % 2].wait()
    copies[_NCHUNK % 2].wait()


def kernel(pos, atom_types):
    del pos
    types = atom_types.reshape(-1)
    types = jnp.pad(types, (0, _TYPES_PAD - _N))
    flat = _onehot_sc(types)
    return flat.reshape(_N, _NUM_TYPES)
